# Initial kernel scaffold; baseline (speedup 1.0000x reference)
import jax
import jax.numpy as jnp
from jax.experimental import pallas as pl


def kernel(input, pos, grid_pos, dx, kernel):
    raise NotImplementedError("write your pallas kernel here")


# speedup vs baseline: 4.7523x; 4.7523x over previous
import functools
import math

import jax
import jax.numpy as jnp
from jax.experimental import pallas as pl

IN_CHANNELS = 32
OUT_CHANNELS = 32


def _ball_to_cube(x, y, z):
    eps = 1e-12
    sq_norm = x * x + y * y + z * z
    small = sq_norm < eps
    sqn_safe = jnp.where(small, 1.0, sq_norm)
    norm = jnp.sqrt(sqn_safe)
    xy2 = x * x + y * y
    cap = (1.25 * z * z) > xy2
    denom_cap = norm + jnp.abs(z)
    s_cap = jnp.sqrt(3.0 * norm / jnp.where(denom_cap < eps, 1.0, denom_cap))
    xy2_safe = jnp.where(xy2 < eps, 1.0, xy2)
    s_side = norm / jnp.sqrt(xy2_safe)
    x1 = jnp.where(cap, x * s_cap, x * s_side)
    y1 = jnp.where(cap, y * s_cap, y * s_side)
    z1 = jnp.where(cap, jnp.sign(z) * norm, 1.5 * z)
    x1 = jnp.where(small, 0.0, x1)
    y1 = jnp.where(small, 0.0, y1)
    z1 = jnp.where(small, 0.0, z1)
    sq_xy = x1 * x1 + y1 * y1
    small_xy = sq_xy < eps
    nxy = jnp.sqrt(jnp.where(small_xy, 1.0, sq_xy))
    condx = jnp.abs(y1) <= jnp.abs(x1)
    dx_safe = jnp.where(jnp.abs(x1) < eps, 1.0, x1)
    dy_safe = jnp.where(jnp.abs(y1) < eps, 1.0, y1)
    tmp_x = jnp.sign(x1) * nxy
    tmp_y = jnp.sign(y1) * nxy
    four_over_pi = 4.0 / math.pi
    rx = jnp.clip(x1 / dy_safe, -1.0, 1.0)
    ry = jnp.clip(y1 / dx_safe, -1.0, 1.0)
    x2 = jnp.where(condx, tmp_x, tmp_y * four_over_pi * _atan(rx))
    y2 = jnp.where(condx, tmp_x * four_over_pi * _atan(ry), tmp_y)
    x2 = jnp.where(small_xy, 0.0, x2)
    y2 = jnp.where(small_xy, 0.0, y2)
    return x2, y2, z1


def _atan(t):
    t2 = t * t
    p = jnp.float32(0.0028662257)
    p = p * t2 - jnp.float32(0.0161657367)
    p = p * t2 + jnp.float32(0.0429096138)
    p = p * t2 - jnp.float32(0.0752896400)
    p = p * t2 + jnp.float32(0.1065626393)
    p = p * t2 - jnp.float32(0.1420889944)
    p = p * t2 + jnp.float32(0.1999355085)
    p = p * t2 - jnp.float32(0.3333314528)
    p = p * t2 + jnp.float32(1.0)
    return p * t


def _interp3(t):
    t = jnp.clip(t, 0.0, 2.0)
    i0 = jnp.clip(jnp.floor(t), 0.0, 1.0)
    f = t - i0
    lo = i0 == 0.0
    w0 = jnp.where(lo, 1.0 - f, 0.0)
    w1 = jnp.where(lo, f, 1.0 - f)
    w2 = jnp.where(lo, 0.0, f)
    return w0, w1, w2


def _dense_body(px_ref, py_ref, pz_ref, gx_ref, gy_ref, gz_ref, feat_ref,
                kflat_ref, out_ref, *, n_tile, g_chunk, n_gchunks, cin):
    px = px_ref[0]
    py = py_ref[0]
    pz = pz_ref[0]

    def chunk(c, carry):
        acc, cnt = carry
        gsl = pl.ds(c * g_chunk, g_chunk)
        gx = gx_ref[0, :, gsl]
        gy = gy_ref[0, :, gsl]
        gz = gz_ref[0, :, gsl]
        featc = feat_ref[0, gsl, :]
        rx = gx - px
        ry = gy - py
        rz = gz - pz
        dist2 = rx * rx + ry * ry + rz * rz
        mask = (dist2 <= 1.0).astype(jnp.float32)
        u, v, w = _ball_to_cube(rx, ry, rz)
        wx = _interp3(u + 1.0)
        wy = _interp3(v + 1.0)
        wz = _interp3(w + 1.0)
        parts = []
        for kz in range(3):
            for ky in range(3):
                wzy = wz[kz] * wy[ky] * mask
                for kx in range(3):
                    wk = wzy * wx[kx]
                    parts.append(
                        jax.lax.dot_general(
                            wk, featc, (((1,), (0,)), ((), ())),
                            preferred_element_type=jnp.float32))
        acc = acc + jnp.concatenate(parts, axis=1)
        cnt = cnt + jnp.sum(mask, axis=1, keepdims=True)
        return acc, cnt

    acc0 = jnp.zeros((n_tile, 27 * cin), jnp.float32)
    cnt0 = jnp.zeros((n_tile, 1), jnp.float32)
    acc, cnt = jax.lax.fori_loop(0, n_gchunks, chunk, (acc0, cnt0))
    out = jax.lax.dot_general(acc, kflat_ref[...], (((1,), (0,)), ((), ())),
                              preferred_element_type=jnp.float32)
    out_ref[0] = out / jnp.maximum(cnt, 1.0)


def kernel(input, pos, grid_pos, dx, kernel, *, interpret=False):
    Bb, cin = input.shape[0], input.shape[1]
    n = pos.shape[1]
    g = grid_pos.shape[0]
    cout = kernel.shape[-1]
    radius = dx * 2.5
    inv_r = 1.0 / radius

    grid_feat = jnp.transpose(input, (0, 2, 3, 4, 1)).reshape(Bb, g, cin)
    kflat = kernel.reshape(27 * cin, cout)

    ps = pos * inv_r
    px = ps[:, :, 0:1]
    py = ps[:, :, 1:2]
    pz = ps[:, :, 2:3]
    gs = (grid_pos * inv_r).T.reshape(1, 3, g)
    gx = gs[:, 0:1, :]
    gy = gs[:, 1:2, :]
    gz = gs[:, 2:3, :]

    n_tile = 256
    g_chunk = 1024
    body = functools.partial(_dense_body, n_tile=n_tile, g_chunk=g_chunk,
                             n_gchunks=g // g_chunk, cin=cin)
    out = pl.pallas_call(
        body,
        grid=(Bb, n // n_tile),
        in_specs=[
            pl.BlockSpec((1, n_tile, 1), lambda b, i: (b, i, 0)),
            pl.BlockSpec((1, n_tile, 1), lambda b, i: (b, i, 0)),
            pl.BlockSpec((1, n_tile, 1), lambda b, i: (b, i, 0)),
            pl.BlockSpec((1, 1, g), lambda b, i: (0, 0, 0)),
            pl.BlockSpec((1, 1, g), lambda b, i: (0, 0, 0)),
            pl.BlockSpec((1, 1, g), lambda b, i: (0, 0, 0)),
            pl.BlockSpec((1, g, cin), lambda b, i: (b, 0, 0)),
            pl.BlockSpec((27 * cin, cout), lambda b, i: (0, 0)),
        ],
        out_specs=pl.BlockSpec((1, n_tile, cout), lambda b, i: (b, i, 0)),
        out_shape=jax.ShapeDtypeStruct((Bb, n, cout), jnp.float32),
        interpret=interpret,
    )(px, py, pz, gx, gy, gz, grid_feat, kflat)
    return out
